# baseline (speedup 1.0000x reference)
import functools

import numpy as np
import jax
import jax.numpy as jnp
from jax import lax
from jax.experimental import pallas as pl
from jax.experimental.pallas import tpu as pltpu
from jax.experimental.pallas import tpu_sc as plsc

NV = 10000; L = 16; ZD = 256; CH = 64; SEG = 9; FD = 8; NL = 6; NH = 8
DK = 64; DV = 64; G = 8; DFF = 256
R = SEG * NV

_P128 = np.array([(j % 8) * 16 + j // 8 for j in range(128)])
_P64 = np.array([(j % 8) * 8 + j // 8 for j in range(64)])
_P512 = np.array([(j % 8) * 64 + j // 8 for j in range(512)])
_PROWS = (np.arange(L)[:, None] * 64 + _P64[None, :]).reshape(-1)

TILE_A = 2000
TILE_B = 1000
TILE_C = 400
SC_CHUNK = 600


def _fold(x, k):
    for _ in range(k):
        w = x.shape[-1] // 2
        x = x[..., :w] + x[..., w:]
    return x


def _unfold(x, k):
    for _ in range(k):
        x = jnp.concatenate([x, x], axis=-1)
    return x


def _elu(x):
    return jnp.where(x > 0, x, jnp.exp(jnp.minimum(x, 0.0)) - 1.0)


def _gn(x, nf, gamma, beta):
    n = float(2 ** nf)
    s = _fold(x, nf)
    s2 = _fold(x * x, nf)
    m = s * (1.0 / n)
    v = s2 * (1.0 / n) - m * m
    inv = lax.rsqrt(v + 1e-5)
    return (x - _unfold(m, nf)) * _unfold(inv, nf) * gamma + beta


def _ln(x, gamma, beta):
    m = jnp.mean(x, axis=-1, keepdims=True)
    v = jnp.mean(x * x, axis=-1, keepdims=True) - m * m
    return (x - m) * lax.rsqrt(v + 1e-5) * gamma + beta


def _dot(a, b):
    return jnp.dot(a, b, preferred_element_type=jnp.float32)


def _pack_tbl(h):
    return jnp.concatenate([h, jnp.zeros_like(h)], axis=1)


def _pre_math(v, x, W1, b1, g1, be1, W2, b2, g2, be2):
    xw = _dot(x, W1[8:])
    h = _dot(v, W1[:8]) + xw + b1
    h = _elu(_gn(h, 4, g1, be1))
    h = _dot(h, W2) + b2
    return _elu(_gn(h, 3, g2, be2))


def _pre_body(v_ref, x_ref, W1_ref, b1_ref, g1_ref, be1_ref,
              W2_ref, b2_ref, g2_ref, be2_ref, o_ref):
    o_ref[...] = _pack_tbl(
        _pre_math(v_ref[...], x_ref[...], W1_ref[...], b1_ref[...],
                  g1_ref[...], be1_ref[...], W2_ref[...],
                  b2_ref[...], g2_ref[...], be2_ref[...]))


def _spiral_math(gth, Ws, bs, gs, bes):
    h = _dot(gth.astype(jnp.bfloat16), Ws) + bs
    return _elu(_gn(h, 3, gs, bes))


def _spiral_body_tbl(g_ref, Ws_ref, bs_ref, gs_ref, bes_ref, o_ref):
    o_ref[...] = _pack_tbl(_spiral_math(g_ref[...], Ws_ref[...], bs_ref[...],
                                        gs_ref[...], bes_ref[...]))


def _spiral_body_f32(g_ref, Ws_ref, bs_ref, gs_ref, bes_ref, o_ref):
    o_ref[...] = _spiral_math(g_ref[...], Ws_ref[...], bs_ref[...],
                              gs_ref[...], bes_ref[...])


def _xf_math(h3, pe, Wq, Wk, Wv, Wo, l1g, l1b, Wf1, bf1, Wf2, bf2,
             l2g, l2b, Wfin, bfin):
    T = h3.shape[1]
    bf = jnp.bfloat16
    t = h3.reshape(SEG * T, CH) + pe
    for i in range(NL):
        tb = t.astype(bf)
        q3 = _dot(tb, Wq[i]).reshape(SEG, T, NH * DK)
        k3 = _dot(tb, Wk[i]).reshape(SEG, T, NH * DK)
        v3 = _dot(tb, Wv[i]).reshape(SEG, T, NH * DV)
        scs = [_fold(q3 * k3[sj], 6) * 0.125 for sj in range(SEG)]
        m = scs[0]
        for s in scs[1:]:
            m = jnp.maximum(m, s)
        es = [jnp.exp(s - m) for s in scs]
        den = es[0]
        for e in es[1:]:
            den = den + e
        inv = 1.0 / den
        o3 = _unfold(es[0] * inv, 6) * v3[0]
        for sj in range(1, SEG):
            o3 = o3 + _unfold(es[sj] * inv, 6) * v3[sj]
        o = o3.reshape(SEG * T, NH * DV)
        t = _ln(t + _dot(o.astype(bf), Wo[i]), l1g[i], l1b[i])
        f = jnp.maximum(_dot(t.astype(bf), Wf1[i]) + bf1[i], 0.0)
        f = _dot(f.astype(bf), Wf2[i]) + bf2[i]
        t = _ln(t + f, l2g[i], l2b[i])
    out = _dot(t, Wfin) + bfin
    return out.reshape(SEG, T, FD)


def _xf_body(h_ref, pe_ref, Wq_ref, Wk_ref, Wv_ref, Wo_ref, l1g_ref, l1b_ref,
             Wf1_ref, bf1_ref, Wf2_ref, bf2_ref, l2g_ref, l2b_ref,
             Wf_ref, bf_ref, o_ref):
    o_ref[...] = _xf_math(h_ref[...], pe_ref[...], Wq_ref[...], Wk_ref[...],
                          Wv_ref[...], Wo_ref[...], l1g_ref[...], l1b_ref[...],
                          Wf1_ref[...], bf1_ref[...], Wf2_ref[...],
                          bf2_ref[...], l2g_ref[...], l2b_ref[...],
                          Wf_ref[...], bf_ref[...])


def _sc_gather(table, idx):
    nw = 32
    b_per_w = idx.shape[0] // nw
    mesh = plsc.VectorSubcoreMesh(core_axis_name="c", subcore_axis_name="s")

    @functools.partial(
        pl.kernel, mesh=mesh,
        out_type=jax.ShapeDtypeStruct((idx.shape[0], 128), jnp.float32),
        scratch_types=[pltpu.VMEM((SC_CHUNK,), jnp.int32),
                       pltpu.VMEM((SC_CHUNK, 128), jnp.float32),
                       pltpu.SemaphoreType.DMA])
    def k(table_hbm, idx_hbm, out_hbm, idx_v, rows_v, sem):
        wid = lax.axis_index("s") * 2 + lax.axis_index("c")
        base = wid * b_per_w

        @pl.loop(0, b_per_w, step=SC_CHUNK)
        def _(off):
            pltpu.sync_copy(idx_hbm.at[pl.ds(base + off, SC_CHUNK)], idx_v)
            pltpu.async_copy(table_hbm.at[idx_v], rows_v, sem).wait()
            pltpu.sync_copy(rows_v, out_hbm.at[pl.ds(base + off, SC_CHUNK)])

    return k(table, idx)


def _full(shape):
    nd = len(shape)
    return pl.BlockSpec(shape, lambda i: (0,) * nd)


def kernel(x, vertices, spiral_indices, params):
    p = params
    W1p = jnp.concatenate([jnp.pad(p['W1'][:3], ((0, 5), (0, 0))),
                           p['W1'][3:]], axis=0)[:, _P128]
    b1p = p['b1'][_P128].reshape(1, -1)
    g1p = p['g1'][_P128].reshape(1, -1)
    be1p = p['be1'][_P128].reshape(1, -1)
    W2p = p['W2'][_P128][:, _P64]
    b2p = p['b2'][_P64].reshape(1, -1)
    g2p = p['g2'][_P64].reshape(1, -1)
    be2p = p['be2'][_P64].reshape(1, -1)
    Wsp = [jnp.pad(p['Ws'][i][_PROWS][:, _P64].reshape(L, CH, CH),
                   ((0, 0), (0, 64), (0, 0))).reshape(L * 128, CH)
           .astype(jnp.bfloat16) for i in range(3)]
    bsp = [p['bs'][i][_P64].reshape(1, -1) for i in range(3)]
    gsp = [p['gs'][i][_P64].reshape(1, -1) for i in range(3)]
    besp = [p['bes'][i][_P64].reshape(1, -1) for i in range(3)]
    bft = jnp.bfloat16
    Wq = jnp.stack([lp['Wq'][_P64][:, _P512] for lp in p['layers']]).astype(bft)
    Wk = jnp.stack([lp['Wk'][_P64][:, _P512] for lp in p['layers']]).astype(bft)
    Wv = jnp.stack([lp['Wv'][_P64][:, _P512] for lp in p['layers']]).astype(bft)
    Wo = jnp.stack([lp['Wo'][_P512][:, _P64] for lp in p['layers']]).astype(bft)
    l1g = jnp.stack([lp['ln1g'][_P64] for lp in p['layers']]).reshape(NL, 1, CH)
    l1b = jnp.stack([lp['ln1b'][_P64] for lp in p['layers']]).reshape(NL, 1, CH)
    Wf1 = jnp.stack([lp['Wf1'][_P64] for lp in p['layers']]).astype(bft)
    bf1 = jnp.stack([lp['bf1'] for lp in p['layers']]).reshape(NL, 1, DFF)
    Wf2 = jnp.stack([lp['Wf2'][:, _P64] for lp in p['layers']]).astype(bft)
    bf2 = jnp.stack([lp['bf2'][_P64] for lp in p['layers']]).reshape(NL, 1, CH)
    l2g = jnp.stack([lp['ln2g'][_P64] for lp in p['layers']]).reshape(NL, 1, CH)
    l2b = jnp.stack([lp['ln2b'][_P64] for lp in p['layers']]).reshape(NL, 1, CH)
    Wfp = p['Wf'][_P64]
    bfp = p['bf'].reshape(1, FD)

    pos = np.arange(SEG)[:, None].astype(np.float64)
    i = np.arange(CH)[None, :]
    angle = pos / np.power(10000.0, (2 * (i // 2)) / float(CH))
    pe = np.zeros((SEG, CH))
    pe[:, 0::2] = np.sin(angle[:, 0::2])
    pe[:, 1::2] = np.cos(angle[:, 1::2])
    pe_exp = jnp.asarray(np.repeat(pe[:, _P64], TILE_C, axis=0), jnp.float32)

    v2 = jnp.pad(vertices.reshape(R, 3), ((0, 0), (0, 5)))
    h = pl.pallas_call(
        _pre_body,
        grid=(R // TILE_A,),
        in_specs=[pl.BlockSpec((TILE_A, 8), lambda i: (i, 0)),
                  _full((1, ZD)), _full(W1p.shape), _full((1, 128)),
                  _full((1, 128)), _full((1, 128)), _full((128, CH)),
                  _full((1, CH)), _full((1, CH)), _full((1, CH))],
        out_specs=pl.BlockSpec((TILE_A, 128), lambda i: (i, 0)),
        out_shape=jax.ShapeDtypeStruct((R, 128), jnp.float32),
    )(v2, x, W1p, b1p, g1p, be1p, W2p, b2p, g2p, be2p)

    idx_flat = (jnp.arange(SEG, dtype=jnp.int32)[:, None, None] * NV
                + spiral_indices.astype(jnp.int32)[None]).reshape(-1)
    for i in range(3):
        gth = _sc_gather(h, idx_flat).reshape(R, L * 128)
        body = _spiral_body_tbl if i < 2 else _spiral_body_f32
        oshape = ((R, 128), jnp.float32) if i < 2 else ((R, CH), jnp.float32)
        h = pl.pallas_call(
            body,
            grid=(R // TILE_B,),
            in_specs=[pl.BlockSpec((TILE_B, L * 128), lambda i: (i, 0)),
                      _full((L * 128, CH)), _full((1, CH)), _full((1, CH)),
                      _full((1, CH))],
            out_specs=pl.BlockSpec((TILE_B, oshape[0][1]), lambda i: (i, 0)),
            out_shape=jax.ShapeDtypeStruct(*oshape),
        )(gth, Wsp[i], bsp[i], gsp[i], besp[i])

    h3 = h.reshape(SEG, NV, CH)
    out = pl.pallas_call(
        _xf_body,
        grid=(NV // TILE_C,),
        in_specs=[pl.BlockSpec((SEG, TILE_C, CH), lambda i: (0, i, 0)),
                  _full(pe_exp.shape), _full(Wq.shape), _full(Wk.shape),
                  _full(Wv.shape), _full(Wo.shape), _full(l1g.shape),
                  _full(l1b.shape), _full(Wf1.shape), _full(bf1.shape),
                  _full(Wf2.shape), _full(bf2.shape), _full(l2g.shape),
                  _full(l2b.shape), _full(Wfp.shape), _full(bfp.shape)],
        out_specs=pl.BlockSpec((SEG, TILE_C, FD), lambda i: (0, i, 0)),
        out_shape=jax.ShapeDtypeStruct((SEG, NV, FD), jnp.float32),
    )(h3, pe_exp, Wq, Wk, Wv, Wo, l1g, l1b, Wf1, bf1, Wf2, bf2,
      l2g, l2b, Wfp, bfp)

    return out.reshape(1, SEG, NV, FD)


# speedup vs baseline: 1.3258x; 1.3258x over previous
import functools

import numpy as np
import jax
import jax.numpy as jnp
from jax import lax
from jax.experimental import pallas as pl
from jax.experimental.pallas import tpu as pltpu
from jax.experimental.pallas import tpu_sc as plsc

NV = 10000; L = 16; ZD = 256; CH = 64; SEG = 9; FD = 8; NL = 6; NH = 8
DK = 64; DV = 64; G = 8; DFF = 256
R = SEG * NV

_P128 = np.array([(j % 8) * 16 + j // 8 for j in range(128)])
_P64 = np.array([(j % 8) * 8 + j // 8 for j in range(64)])
_P512 = np.array([(j % 8) * 64 + j // 8 for j in range(512)])
_PROWS = (np.arange(L)[:, None] * 64 + _P64[None, :]).reshape(-1)

TILE_A = 400
TILE_B = 80
TILE_C = 400
SC_CHUNK = 200
TW = 640


def _fold(x, k):
    for _ in range(k):
        w = x.shape[-1] // 2
        x = x[..., :w] + x[..., w:]
    return x


def _unfold(x, k):
    for _ in range(k):
        x = jnp.concatenate([x, x], axis=-1)
    return x


def _elu(x):
    return jnp.where(x > 0, x, jnp.exp(jnp.minimum(x, 0.0)) - 1.0)


def _gn(x, nf, gamma, beta):
    n = float(2 ** nf)
    s = _fold(x, nf)
    s2 = _fold(x * x, nf)
    m = s * (1.0 / n)
    v = s2 * (1.0 / n) - m * m
    inv = lax.rsqrt(v + 1e-5)
    return (x - _unfold(m, nf)) * _unfold(inv, nf) * gamma + beta


def _ln(x, gamma, beta):
    m = jnp.mean(x, axis=-1, keepdims=True)
    v = jnp.mean(x * x, axis=-1, keepdims=True) - m * m
    return (x - m) * lax.rsqrt(v + 1e-5) * gamma + beta


def _dot(a, b):
    return jnp.dot(a, b, preferred_element_type=jnp.float32)


def _pre_math(v, x, W1, b1, g1, be1, W2, b2, g2, be2):
    xw = _dot(x, W1[8:])
    h = _dot(v, W1[:8]) + xw + b1
    h = _elu(_gn(h, 4, g1, be1))
    h = _dot(h, W2) + b2
    return _elu(_gn(h, 3, g2, be2))


def _pre_body(v_ref, x_ref, W1_ref, b1_ref, g1_ref, be1_ref,
              W2_ref, b2_ref, g2_ref, be2_ref, o_ref):
    v = v_ref[...].reshape(SEG * TILE_A, 8)
    y = _pre_math(v, x_ref[...], W1_ref[...], b1_ref[...], g1_ref[...],
                  be1_ref[...], W2_ref[...], b2_ref[...], g2_ref[...],
                  be2_ref[...])
    parts = [y[s * TILE_A:(s + 1) * TILE_A] for s in range(SEG)]
    parts.append(jnp.zeros((TILE_A, TW - SEG * CH), y.dtype))
    o_ref[...] = jnp.concatenate(parts, axis=-1)


def _spiral_math(gth, Ws, bs, gs, bes):
    h = _dot(gth.astype(jnp.bfloat16), Ws) + bs
    return _elu(_gn(h, 3, gs, bes))


def _spiral_cols(g):
    xs = []
    for s in range(SEG):
        parts = [g[l, :, s * CH:(s + 1) * CH] for l in range(L)]
        xs.append(jnp.concatenate(parts, axis=-1))
    return jnp.concatenate(xs, axis=0)


def _spiral_body_tbl(g_ref, Ws_ref, bs_ref, gs_ref, bes_ref, o_ref):
    y = _spiral_math(_spiral_cols(g_ref[...]), Ws_ref[...], bs_ref[...],
                     gs_ref[...], bes_ref[...])
    T = y.shape[0] // SEG
    parts = [y[s * T:(s + 1) * T] for s in range(SEG)]
    parts.append(jnp.zeros((T, CH), y.dtype))
    o_ref[...] = jnp.concatenate(parts, axis=-1)


def _spiral_body_f32(g_ref, Ws_ref, bs_ref, gs_ref, bes_ref, o_ref):
    y = _spiral_math(_spiral_cols(g_ref[...]), Ws_ref[...], bs_ref[...],
                     gs_ref[...], bes_ref[...])
    o_ref[...] = y.reshape(SEG, y.shape[0] // SEG, CH)


def _xf_math(h3, pe, Wq, Wk, Wv, Wo, l1g, l1b, Wf1, bf1, Wf2, bf2,
             l2g, l2b, Wfin, bfin):
    T = h3.shape[1]
    bf = jnp.bfloat16
    t = h3.reshape(SEG * T, CH) + pe
    for i in range(NL):
        tb = t.astype(bf)
        q3 = _dot(tb, Wq[i]).reshape(SEG, T, NH * DK)
        k3 = _dot(tb, Wk[i]).reshape(SEG, T, NH * DK)
        v3 = _dot(tb, Wv[i]).reshape(SEG, T, NH * DV)
        scs = [_fold(q3 * k3[sj], 6) * 0.125 for sj in range(SEG)]
        m = scs[0]
        for s in scs[1:]:
            m = jnp.maximum(m, s)
        es = [jnp.exp(s - m) for s in scs]
        den = es[0]
        for e in es[1:]:
            den = den + e
        inv = 1.0 / den
        o3 = _unfold(es[0] * inv, 6) * v3[0]
        for sj in range(1, SEG):
            o3 = o3 + _unfold(es[sj] * inv, 6) * v3[sj]
        o = o3.reshape(SEG * T, NH * DV)
        t = _ln(t + _dot(o.astype(bf), Wo[i]), l1g[i], l1b[i])
        f = jnp.maximum(_dot(t.astype(bf), Wf1[i]) + bf1[i], 0.0)
        f = _dot(f.astype(bf), Wf2[i]) + bf2[i]
        t = _ln(t + f, l2g[i], l2b[i])
    out = _dot(t, Wfin) + bfin
    return out.reshape(SEG, T, FD)


def _xf_body(h_ref, pe_ref, Wq_ref, Wk_ref, Wv_ref, Wo_ref, l1g_ref, l1b_ref,
             Wf1_ref, bf1_ref, Wf2_ref, bf2_ref, l2g_ref, l2b_ref,
             Wf_ref, bf_ref, o_ref):
    o_ref[...] = _xf_math(h_ref[...], pe_ref[...], Wq_ref[...], Wk_ref[...],
                          Wv_ref[...], Wo_ref[...], l1g_ref[...], l1b_ref[...],
                          Wf1_ref[...], bf1_ref[...], Wf2_ref[...],
                          bf2_ref[...], l2g_ref[...], l2b_ref[...],
                          Wf_ref[...], bf_ref[...])


def _sc_gather(table, idx):
    nw = 32
    b_per_w = idx.shape[0] // nw
    mesh = plsc.VectorSubcoreMesh(core_axis_name="c", subcore_axis_name="s")

    @functools.partial(
        pl.kernel, mesh=mesh,
        out_type=jax.ShapeDtypeStruct((idx.shape[0], TW), jnp.float32),
        scratch_types=[pltpu.VMEM((SC_CHUNK,), jnp.int32),
                       pltpu.VMEM((SC_CHUNK, TW), jnp.float32),
                       pltpu.SemaphoreType.DMA])
    def k(table_hbm, idx_hbm, out_hbm, idx_v, rows_v, sem):
        wid = lax.axis_index("s") * 2 + lax.axis_index("c")
        base = wid * b_per_w

        @pl.loop(0, b_per_w, step=SC_CHUNK)
        def _(off):
            pltpu.sync_copy(idx_hbm.at[pl.ds(base + off, SC_CHUNK)], idx_v)
            pltpu.async_copy(table_hbm.at[idx_v], rows_v, sem).wait()
            pltpu.sync_copy(rows_v, out_hbm.at[pl.ds(base + off, SC_CHUNK)])

    return k(table, idx)


def _full(shape):
    nd = len(shape)
    return pl.BlockSpec(shape, lambda *_: (0,) * nd)


def kernel(x, vertices, spiral_indices, params):
    p = params
    W1p = jnp.concatenate([jnp.pad(p['W1'][:3], ((0, 5), (0, 0))),
                           p['W1'][3:]], axis=0)[:, _P128]
    b1p = p['b1'][_P128].reshape(1, -1)
    g1p = p['g1'][_P128].reshape(1, -1)
    be1p = p['be1'][_P128].reshape(1, -1)
    W2p = p['W2'][_P128][:, _P64]
    b2p = p['b2'][_P64].reshape(1, -1)
    g2p = p['g2'][_P64].reshape(1, -1)
    be2p = p['be2'][_P64].reshape(1, -1)
    Wsp = [p['Ws'][i][_PROWS][:, _P64].astype(jnp.bfloat16) for i in range(3)]
    bsp = [p['bs'][i][_P64].reshape(1, -1) for i in range(3)]
    gsp = [p['gs'][i][_P64].reshape(1, -1) for i in range(3)]
    besp = [p['bes'][i][_P64].reshape(1, -1) for i in range(3)]
    bft = jnp.bfloat16
    Wq = jnp.stack([lp['Wq'][_P64][:, _P512] for lp in p['layers']]).astype(bft)
    Wk = jnp.stack([lp['Wk'][_P64][:, _P512] for lp in p['layers']]).astype(bft)
    Wv = jnp.stack([lp['Wv'][_P64][:, _P512] for lp in p['layers']]).astype(bft)
    Wo = jnp.stack([lp['Wo'][_P512][:, _P64] for lp in p['layers']]).astype(bft)
    l1g = jnp.stack([lp['ln1g'][_P64] for lp in p['layers']]).reshape(NL, 1, CH)
    l1b = jnp.stack([lp['ln1b'][_P64] for lp in p['layers']]).reshape(NL, 1, CH)
    Wf1 = jnp.stack([lp['Wf1'][_P64] for lp in p['layers']]).astype(bft)
    bf1 = jnp.stack([lp['bf1'] for lp in p['layers']]).reshape(NL, 1, DFF)
    Wf2 = jnp.stack([lp['Wf2'][:, _P64] for lp in p['layers']]).astype(bft)
    bf2 = jnp.stack([lp['bf2'][_P64] for lp in p['layers']]).reshape(NL, 1, CH)
    l2g = jnp.stack([lp['ln2g'][_P64] for lp in p['layers']]).reshape(NL, 1, CH)
    l2b = jnp.stack([lp['ln2b'][_P64] for lp in p['layers']]).reshape(NL, 1, CH)
    Wfp = p['Wf'][_P64]
    bfp = p['bf'].reshape(1, FD)

    pos = np.arange(SEG)[:, None].astype(np.float64)
    i = np.arange(CH)[None, :]
    angle = pos / np.power(10000.0, (2 * (i // 2)) / float(CH))
    pe = np.zeros((SEG, CH))
    pe[:, 0::2] = np.sin(angle[:, 0::2])
    pe[:, 1::2] = np.cos(angle[:, 1::2])
    pe_exp = jnp.asarray(np.repeat(pe[:, _P64], TILE_C, axis=0), jnp.float32)

    v3 = jnp.pad(vertices.reshape(SEG, NV, 3), ((0, 0), (0, 0), (0, 5)))
    h = pl.pallas_call(
        _pre_body,
        grid=(NV // TILE_A,),
        in_specs=[pl.BlockSpec((SEG, TILE_A, 8), lambda j: (0, j, 0)),
                  _full((1, ZD)), _full(W1p.shape), _full((1, 128)),
                  _full((1, 128)), _full((1, 128)), _full((128, CH)),
                  _full((1, CH)), _full((1, CH)), _full((1, CH))],
        out_specs=pl.BlockSpec((TILE_A, TW), lambda j: (j, 0)),
        out_shape=jax.ShapeDtypeStruct((NV, TW), jnp.float32),
    )(v3, x, W1p, b1p, g1p, be1p, W2p, b2p, g2p, be2p)

    idx_flat = spiral_indices.astype(jnp.int32).T.reshape(-1)
    for i in range(3):
        gth = _sc_gather(h, idx_flat).reshape(L, NV, TW)
        body = _spiral_body_tbl if i < 2 else _spiral_body_f32
        if i < 2:
            ospec = pl.BlockSpec((TILE_B, TW), lambda j: (j, 0))
            oshape = jax.ShapeDtypeStruct((NV, TW), jnp.float32)
        else:
            ospec = pl.BlockSpec((SEG, TILE_B, CH), lambda j: (0, j, 0))
            oshape = jax.ShapeDtypeStruct((SEG, NV, CH), jnp.float32)
        h = pl.pallas_call(
            body,
            grid=(NV // TILE_B,),
            in_specs=[pl.BlockSpec((L, TILE_B, TW), lambda j: (0, j, 0)),
                      _full((L * CH, CH)), _full((1, CH)), _full((1, CH)),
                      _full((1, CH))],
            out_specs=ospec,
            out_shape=oshape,
        )(gth, Wsp[i], bsp[i], gsp[i], besp[i])

    h3 = h
    out = pl.pallas_call(
        _xf_body,
        grid=(NV // TILE_C,),
        in_specs=[pl.BlockSpec((SEG, TILE_C, CH), lambda i: (0, i, 0)),
                  _full(pe_exp.shape), _full(Wq.shape), _full(Wk.shape),
                  _full(Wv.shape), _full(Wo.shape), _full(l1g.shape),
                  _full(l1b.shape), _full(Wf1.shape), _full(bf1.shape),
                  _full(Wf2.shape), _full(bf2.shape), _full(l2g.shape),
                  _full(l2b.shape), _full(Wfp.shape), _full(bfp.shape)],
        out_specs=pl.BlockSpec((SEG, TILE_C, FD), lambda i: (0, i, 0)),
        out_shape=jax.ShapeDtypeStruct((SEG, NV, FD), jnp.float32),
    )(h3, pe_exp, Wq, Wk, Wv, Wo, l1g, l1b, Wf1, bf1, Wf2, bf2,
      l2g, l2b, Wfp, bfp)

    return out.reshape(1, SEG, NV, FD)
